# Initial kernel scaffold; baseline (speedup 1.0000x reference)
import functools

import jax
import jax.numpy as jnp
from jax import lax
from jax.experimental import pallas as pl
from jax.experimental.pallas import tpu as pltpu
from jax.experimental.pallas import tpu_sc as plsc

NC = 2
NS = 16
CHUNK = 80


@functools.partial(jax.jit, static_argnames=("with_e",))
def _sc_agg(x, src, dst, edge_attr, zeros_d, zeros_e, with_e=False):
    N, D = x.shape
    E = src.shape[0]
    NW = NC * NS
    epw = E // NW
    n_chunks = epw // CHUNK
    rpt = N // NS

    mesh = plsc.VectorSubcoreMesh(core_axis_name="c", subcore_axis_name="s",
                                  num_cores=NC, num_subcores=NS)

    out_type = [jax.ShapeDtypeStruct((NC, N, D), jnp.float32)]
    scratch = [
        pltpu.VMEM_SHARED((N, D), jnp.float32),
        pltpu.VMEM((CHUNK,), jnp.int32),
        pltpu.VMEM((CHUNK,), jnp.int32),
        pltpu.VMEM((CHUNK, D), jnp.float32),
        pltpu.SemaphoreType.DMA,
    ]
    if with_e:
        ED = edge_attr.shape[1]
        out_type.append(jax.ShapeDtypeStruct((NC, N, ED), jnp.float32))
        scratch += [
            pltpu.VMEM_SHARED((N, ED), jnp.float32),
            pltpu.VMEM((CHUNK, ED), jnp.float32),
        ]

    def body(*refs):
        if with_e:
            (x_hbm, src_hbm, dst_hbm, ea_hbm, zd_hbm, ze_hbm,
             out_hbm, eout_hbm, acc, idx_s, idx_d, rows, sem, eacc, erows) = refs
        else:
            (x_hbm, src_hbm, dst_hbm, zd_hbm,
             out_hbm, acc, idx_s, idx_d, rows, sem) = refs
        c = lax.axis_index("c")
        s = lax.axis_index("s")
        r0 = s * rpt
        pltpu.sync_copy(zd_hbm.at[pl.ds(r0, rpt)], acc.at[pl.ds(r0, rpt)])
        if with_e:
            pltpu.sync_copy(ze_hbm.at[pl.ds(r0, rpt)], eacc.at[pl.ds(r0, rpt)])
        plsc.subcore_barrier()

        base0 = (c * NS + s) * epw

        def step(g, carry):
            base = base0 + g * CHUNK
            pltpu.sync_copy(src_hbm.at[pl.ds(base, CHUNK)], idx_s)
            pltpu.sync_copy(dst_hbm.at[pl.ds(base, CHUNK)], idx_d)
            pltpu.async_copy(x_hbm.at[idx_s], rows, sem).wait()
            pltpu.sync_copy(rows, acc.at[idx_d], add=True)
            if with_e:
                pltpu.sync_copy(ea_hbm.at[pl.ds(base, CHUNK)], erows)
                pltpu.sync_copy(erows, eacc.at[idx_d], add=True)
            return carry

        lax.fori_loop(0, n_chunks, step, 0)
        plsc.subcore_barrier()
        pltpu.sync_copy(acc.at[pl.ds(r0, rpt)], out_hbm.at[c, pl.ds(r0, rpt)])
        if with_e:
            pltpu.sync_copy(eacc.at[pl.ds(r0, rpt)],
                            eout_hbm.at[c, pl.ds(r0, rpt)])

    k = pl.kernel(body, out_type=tuple(out_type), mesh=mesh,
                  scratch_types=tuple(scratch))
    if with_e:
        return k(x, src, dst, edge_attr, zeros_d, zeros_e)
    return k(x, src, dst, zeros_d)


def _tc_gru(Ap, Egp, x, WxT, WeT, WihT, WhhT, bih, bhh):
    N, D = x.shape
    ED = Egp.shape[2]
    BN = 2000
    grid = (N // BN,)

    def body(ap, egp, xr, wxt, wet, wiht, whht, bi, bh, o):
        A = ap[0] + ap[1]
        Eg = egp[0] + egp[1]
        m = (jnp.dot(A, wxt[...], preferred_element_type=jnp.float32)
             + jnp.dot(Eg, wet[...], preferred_element_type=jnp.float32))
        gi = jnp.dot(m, wiht[...], preferred_element_type=jnp.float32) + bi[...]
        gh = jnp.dot(xr[...], whht[...], preferred_element_type=jnp.float32) + bh[...]
        r = jax.nn.sigmoid(gi[:, :D] + gh[:, :D])
        z = jax.nn.sigmoid(gi[:, D:2 * D] + gh[:, D:2 * D])
        n = jnp.tanh(gi[:, 2 * D:] + r * gh[:, 2 * D:])
        o[...] = (1.0 - z) * n + z * xr[...]

    return pl.pallas_call(
        body,
        grid=grid,
        in_specs=[
            pl.BlockSpec((NC, BN, D), lambda i: (0, i, 0)),
            pl.BlockSpec((NC, BN, ED), lambda i: (0, i, 0)),
            pl.BlockSpec((BN, D), lambda i: (i, 0)),
            pl.BlockSpec((D, D), lambda i: (0, 0)),
            pl.BlockSpec((ED, D), lambda i: (0, 0)),
            pl.BlockSpec((D, 3 * D), lambda i: (0, 0)),
            pl.BlockSpec((D, 3 * D), lambda i: (0, 0)),
            pl.BlockSpec((1, 3 * D), lambda i: (0, 0)),
            pl.BlockSpec((1, 3 * D), lambda i: (0, 0)),
        ],
        out_specs=pl.BlockSpec((BN, D), lambda i: (i, 0)),
        out_shape=jax.ShapeDtypeStruct((N, D), jnp.float32),
    )(Ap, Egp, x, WxT, WeT, WihT, WhhT, bih, bhh)


def kernel(x, edge_index, edge_attr, W_msg1, W_ih1, W_hh1, b_ih1, b_hh1,
           W_msg2, W_ih2, W_hh2, b_ih2, b_hh2):
    N, D = x.shape
    ED = edge_attr.shape[1]
    src = edge_index[0]
    dst = edge_index[1]
    zeros_d = jnp.zeros((N, D), jnp.float32)
    zeros_e = jnp.zeros((N, ED), jnp.float32)

    def layer(h, W_msg, W_ih, W_hh, b_ih, b_hh, Egp, first):
        if first:
            Ap, Egp = _sc_agg(h, src, dst, edge_attr, zeros_d, zeros_e,
                              with_e=True)
        else:
            (Ap,) = _sc_agg(h, src, dst, edge_attr, zeros_d, zeros_e,
                            with_e=False)
        WxT = W_msg[:, :D].T
        WeT = W_msg[:, D:].T
        h1 = _tc_gru(Ap, Egp, h, WxT, WeT, W_ih.T, W_hh.T,
                     b_ih.reshape(1, -1), b_hh.reshape(1, -1))
        return h1, Egp

    x1, Egp = layer(x, W_msg1, W_ih1, W_hh1, b_ih1, b_hh1, None, True)
    x2, _ = layer(x1, W_msg2, W_ih2, W_hh2, b_ih2, b_hh2, Egp, False)
    return x2


# speedup vs baseline: 3.7326x; 3.7326x over previous
import jax
import jax.numpy as jnp
from jax import lax
from jax.experimental import pallas as pl
from jax.experimental.pallas import tpu as pltpu
from jax.experimental.pallas import tpu_sc as plsc

NC = 2
NS = 16
CHUNK = 80
CR = 64
D = 128


def _seg_rows(num_segments):
    per_tile = -(-num_segments // NS)
    return -(-per_tile // CR) * CR


def _sc_segsum(table, dst, num_segments, src=None):
    E = dst.shape[0]
    NW = NC * NS
    epw = E // NW
    n_chunks = epw // CHUNK
    rpt = _seg_rows(num_segments)
    NP = NS * rpt

    mesh = plsc.VectorSubcoreMesh(core_axis_name="c", subcore_axis_name="s",
                                  num_cores=NC, num_subcores=NS)

    scratch = [
        pltpu.VMEM_SHARED((NP, D), jnp.float32),
        pltpu.VMEM((CHUNK,), jnp.int32),
        pltpu.VMEM((CHUNK, D), jnp.float32),
        pltpu.VMEM((CR, D), jnp.float32),
        pltpu.SemaphoreType.DMA,
    ]
    if src is not None:
        scratch.insert(1, pltpu.VMEM((CHUNK,), jnp.int32))

    def body(*refs):
        if src is not None:
            (table_hbm, src_hbm, dst_hbm,
             out_hbm, acc, idx_s, idx_d, rows, stage, sem) = refs
        else:
            (table_hbm, dst_hbm,
             out_hbm, acc, idx_d, rows, stage, sem) = refs
        c = lax.axis_index("c")
        s = lax.axis_index("s")
        r0 = pl.multiple_of(s * rpt, 8)

        zv = jnp.zeros((16,), jnp.float32)

        def zrow(r, carry):
            for j in range(D // 16):
                stage[r, pl.ds(j * 16, 16)] = zv
            return carry

        lax.fori_loop(0, CR, zrow, 0)

        def zcopy(k, carry):
            rk = pl.multiple_of(r0 + k * CR, 8)
            pltpu.sync_copy(stage, acc.at[pl.ds(rk, CR)])
            return carry

        lax.fori_loop(0, rpt // CR, zcopy, 0)
        plsc.subcore_barrier()

        base0 = (c * NS + s) * epw

        def step(g, carry):
            base = pl.multiple_of(base0 + g * CHUNK, 8)
            pltpu.sync_copy(dst_hbm.at[pl.ds(base, CHUNK)], idx_d)
            if src is not None:
                pltpu.sync_copy(src_hbm.at[pl.ds(base, CHUNK)], idx_s)
                pltpu.async_copy(table_hbm.at[idx_s], rows, sem).wait()
            else:
                pltpu.sync_copy(table_hbm.at[pl.ds(base, CHUNK)], rows)
            pltpu.sync_copy(rows, acc.at[idx_d], add=True)
            return carry

        lax.fori_loop(0, n_chunks, step, 0)
        plsc.subcore_barrier()

        w0 = pl.multiple_of(c * NP + r0, 8)

        def wcopy(k, carry):
            rk = pl.multiple_of(r0 + k * CR, 8)
            wk = pl.multiple_of(w0 + k * CR, 8)
            pltpu.sync_copy(acc.at[pl.ds(rk, CR)], stage)
            pltpu.sync_copy(stage, out_hbm.at[pl.ds(wk, CR)])
            return carry

        lax.fori_loop(0, rpt // CR, wcopy, 0)

    k = pl.kernel(body,
                  out_type=jax.ShapeDtypeStruct((NC * NP, D), jnp.float32),
                  mesh=mesh, scratch_types=tuple(scratch))
    if src is not None:
        return k(table, src, dst)
    return k(table, dst)


def _tc_gru(Ap, Egp, x, WxT, WeTp, WihT, WhhT, bih, bhh):
    N, _ = x.shape
    BN = 2000
    grid = (N // BN,)

    def body(ap, egp, xr, wxt, wetp, wiht, whht, bi, bh, o):
        A = ap[0] + ap[1]
        Eg = egp[0] + egp[1]
        m = (jnp.dot(A, wxt[...], preferred_element_type=jnp.float32)
             + jnp.dot(Eg, wetp[...], preferred_element_type=jnp.float32))
        gi = jnp.dot(m, wiht[...], preferred_element_type=jnp.float32) + bi[...]
        gh = jnp.dot(xr[...], whht[...], preferred_element_type=jnp.float32) + bh[...]
        r = jax.nn.sigmoid(gi[:, :D] + gh[:, :D])
        z = jax.nn.sigmoid(gi[:, D:2 * D] + gh[:, D:2 * D])
        n = jnp.tanh(gi[:, 2 * D:] + r * gh[:, 2 * D:])
        o[...] = (1.0 - z) * n + z * xr[...]

    return pl.pallas_call(
        body,
        grid=grid,
        in_specs=[
            pl.BlockSpec((NC, BN, D), lambda i: (0, i, 0)),
            pl.BlockSpec((NC, BN, D), lambda i: (0, i, 0)),
            pl.BlockSpec((BN, D), lambda i: (i, 0)),
            pl.BlockSpec((D, D), lambda i: (0, 0)),
            pl.BlockSpec((D, D), lambda i: (0, 0)),
            pl.BlockSpec((D, 3 * D), lambda i: (0, 0)),
            pl.BlockSpec((D, 3 * D), lambda i: (0, 0)),
            pl.BlockSpec((1, 3 * D), lambda i: (0, 0)),
            pl.BlockSpec((1, 3 * D), lambda i: (0, 0)),
        ],
        out_specs=pl.BlockSpec((BN, D), lambda i: (i, 0)),
        out_shape=jax.ShapeDtypeStruct((N, D), jnp.float32),
    )(Ap, Egp, x, WxT, WeTp, WihT, WhhT, bih, bhh)


def kernel(x, edge_index, edge_attr, W_msg1, W_ih1, W_hh1, b_ih1, b_hh1,
           W_msg2, W_ih2, W_hh2, b_ih2, b_hh2):
    N, _ = x.shape
    ED = edge_attr.shape[1]
    src = edge_index[0]
    dst = edge_index[1]
    NP = NS * _seg_rows(N)

    ea_pad = jnp.pad(edge_attr, ((0, 0), (0, D - ED)))
    Egp = _sc_segsum(ea_pad, dst, N).reshape(NC, NP, D)

    def layer(h, W_msg, W_ih, W_hh, b_ih, b_hh):
        Ap = _sc_segsum(h, dst, N, src=src).reshape(NC, NP, D)
        WxT = W_msg[:, :D].T
        WeTp = jnp.pad(W_msg[:, D:].T, ((0, D - ED), (0, 0)))
        return _tc_gru(Ap, Egp, h, WxT, WeTp, W_ih.T, W_hh.T,
                       b_ih.reshape(1, -1), b_hh.reshape(1, -1))

    x1 = layer(x, W_msg1, W_ih1, W_hh1, b_ih1, b_hh1)
    x2 = layer(x1, W_msg2, W_ih2, W_hh2, b_ih2, b_hh2)
    return x2
